# baseline (speedup 1.0000x reference)
import functools

import jax
import jax.numpy as jnp
from jax import lax
from jax.experimental import pallas as pl
from jax.experimental.pallas import tpu as pltpu
from jax.experimental.pallas import tpu_sc as plsc

EMB = 32
DENSE = 32
BATCH = 16384

NC = 2
NS = 16
NW = NC * NS
B_PER_W = BATCH // NW
CHUNK = 128
NCHUNK = B_PER_W // CHUNK


def _gather_body(idx_hbm, table_hbm, out_hbm, idx_v, rows_v, sem):
    wid = lax.axis_index("s") * NC + lax.axis_index("c")
    base = wid * B_PER_W
    pltpu.sync_copy(idx_hbm.at[wid], idx_v)
    copies = [
        pltpu.async_copy(table_hbm.at[idx_v.at[j]],
                         rows_v.at[pl.ds(j * CHUNK, CHUNK)], sem)
        for j in range(NCHUNK)
    ]
    for c in copies:
        c.wait()
    pltpu.sync_copy(rows_v, out_hbm.at[pl.ds(base, B_PER_W)])


_gather = functools.partial(
    pl.kernel,
    mesh=plsc.VectorSubcoreMesh(core_axis_name="c", subcore_axis_name="s"),
    compiler_params=pltpu.CompilerParams(use_tc_tiling_on_sc=False),
    out_type=jax.ShapeDtypeStruct((BATCH, EMB), jnp.float32),
    scratch_types=[
        pltpu.VMEM((NCHUNK, CHUNK), jnp.int32),
        pltpu.VMEM((B_PER_W, EMB), jnp.float32),
        pltpu.SemaphoreType.DMA,
    ],
)(_gather_body)

ROWS_BLK = 2048


def _proj_body(emb_ref, w_ref, b_ref, out_ref):
    out_ref[...] = (
        jnp.dot(emb_ref[...], w_ref[...], preferred_element_type=jnp.float32)
        + b_ref[...]
    )


def kernel(indices, table, W, b):
    idx3 = indices.astype(jnp.int32).reshape(NW, NCHUNK, CHUNK)
    emb = _gather(idx3, table)
    out = pl.pallas_call(
        _proj_body,
        grid=(BATCH // ROWS_BLK,),
        in_specs=[
            pl.BlockSpec((ROWS_BLK, EMB), lambda i: (i, 0)),
            pl.BlockSpec((EMB, DENSE), lambda i: (0, 0)),
            pl.BlockSpec((1, DENSE), lambda i: (0, 0)),
        ],
        out_specs=pl.BlockSpec((ROWS_BLK, DENSE), lambda i: (i, 0)),
        out_shape=jax.ShapeDtypeStruct((BATCH, DENSE), jnp.float32),
    )(emb, W, b.reshape(1, DENSE))
    return out


# speedup vs baseline: 1.6313x; 1.6313x over previous
import functools

import jax
import jax.numpy as jnp
from jax import lax
from jax.experimental import pallas as pl
from jax.experimental.pallas import tpu as pltpu
from jax.experimental.pallas import tpu_sc as plsc

EMB = 32
DENSE = 32
BATCH = 16384

NC = 2
NS = 16
NW = NC * NS
B_PER_W = BATCH // NW


def _gather_body(idx_hbm, table_hbm, out_hbm, idx_s, rows_v, sem):
    wid = lax.axis_index("s") * NC + lax.axis_index("c")
    base = wid * B_PER_W
    pltpu.sync_copy(idx_hbm.at[wid], idx_s)

    def issue(g, _):
        v = idx_s[pl.ds(g * 16, 16)]
        for k in range(16):
            pltpu.async_copy(table_hbm.at[pl.ds(v[k], 1), :],
                             rows_v.at[pl.ds(g * 16 + k, 1), :], sem)
        return ()

    lax.fori_loop(0, B_PER_W // 16, issue, ())
    pltpu.make_async_copy(out_hbm.at[pl.ds(base, B_PER_W)], rows_v, sem).wait()
    pltpu.sync_copy(rows_v, out_hbm.at[pl.ds(base, B_PER_W)])


_gather = functools.partial(
    pl.kernel,
    mesh=plsc.VectorSubcoreMesh(core_axis_name="c", subcore_axis_name="s"),
    out_type=jax.ShapeDtypeStruct((BATCH, EMB), jnp.float32),
    scratch_types=[
        pltpu.VMEM((B_PER_W,), jnp.int32),
        pltpu.VMEM((B_PER_W, EMB), jnp.float32),
        pltpu.SemaphoreType.DMA,
    ],
)(_gather_body)

ROWS_BLK = 2048


def _proj_body(emb_ref, w_ref, b_ref, out_ref):
    out_ref[...] = (
        jnp.dot(emb_ref[...], w_ref[...], preferred_element_type=jnp.float32)
        + b_ref[...]
    )


def kernel(indices, table, W, b):
    idx2 = indices.astype(jnp.int32).reshape(NW, B_PER_W)
    emb = _gather(idx2, table)
    out = pl.pallas_call(
        _proj_body,
        grid=(BATCH // ROWS_BLK,),
        in_specs=[
            pl.BlockSpec((ROWS_BLK, EMB), lambda i: (i, 0)),
            pl.BlockSpec((EMB, DENSE), lambda i: (0, 0)),
            pl.BlockSpec((1, DENSE), lambda i: (0, 0)),
        ],
        out_specs=pl.BlockSpec((ROWS_BLK, DENSE), lambda i: (i, 0)),
        out_shape=jax.ShapeDtypeStruct((BATCH, DENSE), jnp.float32),
    )(emb, W, b.reshape(1, DENSE))
    return out
